# Initial kernel scaffold; baseline (speedup 1.0000x reference)
import jax
import jax.numpy as jnp
from jax.experimental import pallas as pl


def kernel(z, codebook):
    raise NotImplementedError("write your pallas kernel here")


# speedup vs baseline: 1.7727x; 1.7727x over previous
import functools

import jax
import jax.numpy as jnp
from jax import lax
from jax.experimental import pallas as pl
from jax.experimental.pallas import tpu as pltpu
from jax.experimental.pallas import tpu_sc as plsc

DIM = 32
K = 512

NC = 2
NS = 16
L = 16
NW = NC * NS

TN = 1024


def _argmin_body(z_ref, cb_ref, idx_ref):
    dotp = jnp.dot(z_ref[...], cb_ref[...], preferred_element_type=jnp.float32)
    m = jnp.min(dotp, axis=1, keepdims=True)
    ks = lax.broadcasted_iota(jnp.int32, dotp.shape, 1)
    idx_ref[...] = jnp.min(jnp.where(dotp == m, ks, K), axis=1).astype(jnp.int32)


def _tc_argmin(z2, codebook):
    n = z2.shape[0]
    return pl.pallas_call(
        _argmin_body,
        grid=(n // TN,),
        in_specs=[
            pl.BlockSpec((TN, DIM), lambda i: (i, 0)),
            pl.BlockSpec((DIM, K), lambda i: (0, 0)),
        ],
        out_specs=pl.BlockSpec((TN,), lambda i: (i,)),
        out_shape=jax.ShapeDtypeStruct((n,), jnp.int32),
    )(z2, codebook)


def _make_sc_gather(n):
    c = n // NW
    sub = 2048
    mesh = plsc.VectorSubcoreMesh(core_axis_name="c", subcore_axis_name="s")

    @functools.partial(
        pl.kernel,
        mesh=mesh,
        out_type=jax.ShapeDtypeStruct((DIM * n,), jnp.float32),
        compiler_params=pltpu.CompilerParams(needs_layout_passes=False),
        scratch_types=[
            pltpu.VMEM((c,), jnp.int32),
            pltpu.VMEM((DIM * K,), jnp.float32),
            pltpu.VMEM((DIM * sub,), jnp.float32),
        ],
    )
    def gather_kernel(cb_hbm, idx_hbm, out_hbm, idx_v, cb_v, out_v):
        wid = lax.axis_index("s") * NC + lax.axis_index("c")
        base = wid * c
        pltpu.sync_copy(cb_hbm, cb_v)
        pltpu.sync_copy(idx_hbm.at[pl.ds(base, c)], idx_v)
        for s in range(c // sub):
            @pl.loop(0, sub // L)
            def _(j):
                col = j * L
                iv = idx_v[pl.ds(s * sub + col, L)]
                for d in range(DIM):
                    out_v[pl.ds(d * sub + col, L)] = plsc.load_gather(
                        cb_v, [iv + d * K])
            for d in range(DIM):
                pltpu.sync_copy(
                    out_v.at[pl.ds(d * sub, sub)],
                    out_hbm.at[pl.ds(d * n + base + s * sub, sub)])

    return gather_kernel


def kernel(z, codebook):
    prev_shape = z.shape
    z2 = z.reshape(-1, DIM)
    n = z2.shape[0]
    idx = _tc_argmin(z2, codebook)
    qt = _make_sc_gather(n)(codebook.reshape(-1), idx)
    return qt.reshape(prev_shape)


# speedup vs baseline: 2.7247x; 1.5371x over previous
import functools

import jax
import jax.numpy as jnp
from jax import lax
from jax.experimental import pallas as pl
from jax.experimental.pallas import tpu as pltpu
from jax.experimental.pallas import tpu_sc as plsc

DIM = 32
K = 512

NC = 2
NS = 16
L = 16
NW = NC * NS

TN = 1024


def _argmin_body(z_ref, cb_ref, idx_ref):
    dotp = lax.dot_general(
        cb_ref[...], z_ref[...], (((0,), (1,)), ((), ())),
        preferred_element_type=jnp.float32)
    m = jnp.min(dotp, axis=0, keepdims=True)
    ks = lax.broadcasted_iota(jnp.int32, dotp.shape, 0)
    idx_ref[...] = jnp.min(jnp.where(dotp == m, ks, K), axis=0)


def _tc_argmin(z2, codebook):
    n = z2.shape[0]
    return pl.pallas_call(
        _argmin_body,
        grid=(n // TN,),
        in_specs=[
            pl.BlockSpec((TN, DIM), lambda i: (i, 0)),
            pl.BlockSpec((DIM, K), lambda i: (0, 0)),
        ],
        out_specs=pl.BlockSpec((TN,), lambda i: (i,)),
        out_shape=jax.ShapeDtypeStruct((n,), jnp.int32),
    )(z2, codebook)


def _make_sc_gather(n):
    c = n // NW
    sub = 2048
    mesh = plsc.VectorSubcoreMesh(core_axis_name="c", subcore_axis_name="s")

    @functools.partial(
        pl.kernel,
        mesh=mesh,
        out_type=jax.ShapeDtypeStruct((DIM * n,), jnp.float32),
        compiler_params=pltpu.CompilerParams(needs_layout_passes=False),
        scratch_types=[
            pltpu.VMEM((c,), jnp.int32),
            pltpu.VMEM((DIM * K,), jnp.float32),
            pltpu.VMEM((DIM * sub,), jnp.float32),
        ],
    )
    def gather_kernel(cb_hbm, idx_hbm, out_hbm, idx_v, cb_v, out_v):
        wid = lax.axis_index("s") * NC + lax.axis_index("c")
        base = wid * c
        pltpu.sync_copy(cb_hbm, cb_v)
        pltpu.sync_copy(idx_hbm.at[pl.ds(base, c)], idx_v)
        for s in range(c // sub):
            @pl.loop(0, sub // L)
            def _(j):
                col = j * L
                iv = idx_v[pl.ds(s * sub + col, L)]
                for d in range(DIM):
                    out_v[pl.ds(d * sub + col, L)] = plsc.load_gather(
                        cb_v, [iv + d * K])
            for d in range(DIM):
                pltpu.sync_copy(
                    out_v.at[pl.ds(d * sub, sub)],
                    out_hbm.at[pl.ds(d * n + base + s * sub, sub)])

    return gather_kernel


def kernel(z, codebook):
    prev_shape = z.shape
    z2 = z.reshape(-1, DIM)
    n = z2.shape[0]
    idx = _tc_argmin(z2, codebook)
    qt = _make_sc_gather(n)(codebook.reshape(-1), idx)
    return qt.reshape(prev_shape)
